# baseline (speedup 1.0000x reference)
import functools

import jax
import jax.numpy as jnp
from jax import lax
from jax.experimental import pallas as pl
from jax.experimental.pallas import tpu as pltpu

N = 10000
NFEAT = 128
NHID = 128
BMC = 200
DEPTH = 6
NB = N // BMC


def _in_copy(adj_hbm, bufs, isem, chunk, slot):
    return pltpu.make_async_copy(
        adj_hbm.at[pl.ds(chunk * BMC, BMC), :], bufs.at[slot], isem.at[slot]
    )


def _out_copy(obuf, out_hbm, osem, chunk, oslot):
    return pltpu.make_async_copy(
        obuf.at[oslot], out_hbm.at[pl.ds(chunk * BMC, BMC), :], osem.at[oslot]
    )


def _body(adj_hbm, x_ref, w_ref, b_ref, out_hbm, sup_ref, bufs, obuf, isem, osem):
    for s in range(DEPTH):
        _in_copy(adj_hbm, bufs, isem, s, s).start()

    sup_ref[...] = jnp.dot(
        x_ref[...], w_ref[...], preferred_element_type=jnp.float32
    ).astype(jnp.bfloat16)
    bias = b_ref[...]

    def outer(g, carry):
        del carry
        for s in range(DEPTH):
            i = g * DEPTH + s
            _in_copy(adj_hbm, bufs, isem, i, s).wait()
            if s >= 2:
                _out_copy(obuf, out_hbm, osem, i - 2, s % 2).wait()
            else:

                @pl.when(g > 0)
                def _():
                    _out_copy(obuf, out_hbm, osem, i - 2, s % 2).wait()

            acc = jnp.dot(
                bufs[s].astype(jnp.bfloat16),
                sup_ref[...],
                preferred_element_type=jnp.float32,
            )
            obuf[s % 2] = jax.nn.gelu(acc + bias)
            _out_copy(obuf, out_hbm, osem, i, s % 2).start()

            nxt = i + DEPTH

            @pl.when(nxt < NB)
            def _():
                _in_copy(adj_hbm, bufs, isem, nxt, s).start()

        return 0

    lax.fori_loop(0, NB // DEPTH, outer, 0)

    base = (NB // DEPTH) * DEPTH
    for s in range(NB - base):
        i = base + s
        _in_copy(adj_hbm, bufs, isem, i, s).wait()
        _out_copy(obuf, out_hbm, osem, i - 2, i % 2).wait()
        acc = jnp.dot(
            bufs[s].astype(jnp.bfloat16),
            sup_ref[...],
            preferred_element_type=jnp.float32,
        )
        obuf[i % 2] = jax.nn.gelu(acc + bias)
        _out_copy(obuf, out_hbm, osem, i, i % 2).start()

    _out_copy(obuf, out_hbm, osem, NB - 2, (NB - 2) % 2).wait()
    _out_copy(obuf, out_hbm, osem, NB - 1, (NB - 1) % 2).wait()


def kernel(x, adj, W, b):
    b2 = b.reshape(1, NHID)
    return pl.pallas_call(
        _body,
        in_specs=[
            pl.BlockSpec(memory_space=pl.ANY),
            pl.BlockSpec((N, NFEAT), lambda: (0, 0)),
            pl.BlockSpec((NFEAT, NHID), lambda: (0, 0)),
            pl.BlockSpec((1, NHID), lambda: (0, 0)),
        ],
        out_specs=pl.BlockSpec(memory_space=pl.ANY),
        out_shape=jax.ShapeDtypeStruct((N, NHID), jnp.float32),
        scratch_shapes=[
            pltpu.VMEM((N, NHID), jnp.bfloat16),
            pltpu.VMEM((DEPTH, BMC, N), jnp.float32),
            pltpu.VMEM((2, BMC, NHID), jnp.float32),
            pltpu.SemaphoreType.DMA((DEPTH,)),
            pltpu.SemaphoreType.DMA((2,)),
        ],
        compiler_params=pltpu.CompilerParams(
            vmem_limit_bytes=64 * 1024 * 1024,
        ),
    )(adj, x, W, b2)


# speedup vs baseline: 1.0226x; 1.0226x over previous
import jax
import jax.numpy as jnp
from jax.experimental import pallas as pl
from jax.experimental.pallas import tpu as pltpu

N = 10000
NFEAT = 128
NHID = 128
BM = 400


def _body(adj_ref, x_ref, w_ref, b_ref, out_ref, support_ref):
    @pl.when(pl.program_id(0) == 0)
    def _():
        support_ref[...] = jnp.dot(
            x_ref[...], w_ref[...], preferred_element_type=jnp.float32
        ).astype(jnp.bfloat16)

    acc = jnp.dot(
        adj_ref[...].astype(jnp.bfloat16),
        support_ref[...],
        preferred_element_type=jnp.float32,
    )
    out_ref[...] = jax.nn.gelu(acc + b_ref[...])


def kernel(x, adj, W, b):
    b2 = b.reshape(1, NHID)
    grid = ((N + BM - 1) // BM,)
    return pl.pallas_call(
        _body,
        grid=grid,
        in_specs=[
            pl.BlockSpec((BM, N), lambda i: (i, 0)),
            pl.BlockSpec((N, NFEAT), lambda i: (0, 0)),
            pl.BlockSpec((NFEAT, NHID), lambda i: (0, 0)),
            pl.BlockSpec((1, NHID), lambda i: (0, 0)),
        ],
        out_specs=pl.BlockSpec((BM, NHID), lambda i: (i, 0)),
        out_shape=jax.ShapeDtypeStruct((N, NHID), jnp.float32),
        scratch_shapes=[pltpu.VMEM((N, NHID), jnp.bfloat16)],
        compiler_params=pltpu.CompilerParams(
            vmem_limit_bytes=64 * 1024 * 1024,
        ),
    )(adj, x, W, b2)


# speedup vs baseline: 1.0365x; 1.0136x over previous
import jax
import jax.numpy as jnp
from jax.experimental import pallas as pl
from jax.experimental.pallas import tpu as pltpu

N = 10000
NFEAT = 128
NHID = 128
BM = 400


def _body(adj_ref, x_ref, w_ref, b_ref, out_ref, support_ref):
    @pl.when(pl.program_id(0) == 0)
    def _():
        support_ref[...] = jnp.dot(
            x_ref[...], w_ref[...], preferred_element_type=jnp.float32
        ).astype(jnp.bfloat16)

    acc = jnp.dot(
        adj_ref[...].astype(jnp.bfloat16),
        support_ref[...],
        preferred_element_type=jnp.float32,
    )
    out_ref[...] = jax.nn.gelu(acc + b_ref[...])


def kernel(x, adj, W, b):
    b2 = b.reshape(1, NHID)
    grid = (N // BM,)
    return pl.pallas_call(
        _body,
        grid=grid,
        in_specs=[
            pl.BlockSpec((BM, N), lambda i: (i, 0)),
            pl.BlockSpec((N, NFEAT), lambda i: (0, 0)),
            pl.BlockSpec((NFEAT, NHID), lambda i: (0, 0)),
            pl.BlockSpec((1, NHID), lambda i: (0, 0)),
        ],
        out_specs=pl.BlockSpec((BM, NHID), lambda i: (i, 0)),
        out_shape=jax.ShapeDtypeStruct((N, NHID), jnp.float32),
        scratch_shapes=[pltpu.VMEM((N, NHID), jnp.bfloat16)],
    )(adj, x, W, b2)
